# baseline (speedup 1.0000x reference)
import functools

import jax
import jax.numpy as jnp
import numpy as np
from jax import lax
from jax.experimental import pallas as pl
from jax.experimental.pallas import tpu as pltpu
from jax.experimental.pallas import tpu_sc as plsc

_NUM_EMB = 1024
_EMB_DIM = 64
_COM_COEF = 0.25
_BM = 1024
_NW = 32
_NROWS = 32768
_HALF = _NROWS // 2
_BPW = _HALF // _NW


def _tc_step(x_ref, d_ref):
    xb = x_ref[...]
    dm = d_ref[...]
    sim = lax.dot_general(xb, dm, (((1,), (0,)), ((), ())),
                          preferred_element_type=jnp.float32)
    en2 = jnp.sum(dm * dm, axis=0, keepdims=True)
    dist = en2 - 2.0 * sim
    idx = jnp.argmin(dist, axis=1).astype(jnp.int32)
    m = jnp.min(dist, axis=1, keepdims=True)
    onehot = idx[:, None] == lax.broadcasted_iota(jnp.int32, (_BM, _NUM_EMB), 1)
    encf = onehot.astype(jnp.float32)
    ones_r = jnp.ones((1, _BM), jnp.float32)
    h = lax.dot_general(ones_r, encf, (((1,), (0,)), ((), ())),
                        preferred_element_type=jnp.float32)
    sq = xb * xb
    ones_c64 = jnp.ones((_EMB_DIM, 1), jnp.float32)
    xn2 = lax.dot_general(sq, ones_c64, (((1,), (0,)), ((), ())),
                          preferred_element_type=jnp.float32)
    row_min = m + xn2
    tot = lax.dot_general(ones_r, row_min, (((1,), (0,)), ((), ())),
                          preferred_element_type=jnp.float32)
    return idx, tot, h


def _tc_body_a(x_ref, d_ref, idx_ref, a_ref, h_ref, dt_ref, acc):
    i = pl.program_id(0)
    nsteps = pl.num_programs(0)
    idx, tot, h = _tc_step(x_ref, d_ref)
    idx_ref[...] = idx[:, None]

    @pl.when(i == 0)
    def _():
        acc[0, 0] = 0.0
        h_ref[...] = jnp.zeros_like(h_ref)
        dt_ref[...] = lax.transpose(d_ref[...], (1, 0))

    acc[0, 0] += tot[0, 0]
    h_ref[...] += h

    @pl.when(i == nsteps - 1)
    def _():
        a_ref[...] = jnp.full((1, 1), acc[0, 0], jnp.float32)


def _tc_body_b(x_ref, d_ref, a_in, h_in, idx_ref, loss_ref, perp_ref, hist, acc):
    i = pl.program_id(0)
    nsteps = pl.num_programs(0)
    idx, tot, h = _tc_step(x_ref, d_ref)
    idx_ref[...] = idx[:, None]

    @pl.when(i == 0)
    def _():
        acc[0, 0] = a_in[0, 0]
        hist[...] = h_in[...]

    acc[0, 0] += tot[0, 0]
    hist[...] += h

    @pl.when(i == nsteps - 1)
    def _():
        loss = (1.0 + _COM_COEF) * acc[0, 0] / (_NROWS * _EMB_DIM)
        loss_ref[...] = jnp.full((1, 1), loss, jnp.float32)
        p = hist[...] / _NROWS
        perp = jnp.exp(-jnp.sum(p * jnp.log(p + 1e-10)))
        perp_ref[...] = jnp.full((1, 1), perp, jnp.float32)


def _tc_half_a(xh, dictionary):
    grid = _HALF // _BM
    return pl.pallas_call(
        _tc_body_a,
        grid=(grid,),
        in_specs=[
            pl.BlockSpec((_BM, _EMB_DIM), lambda i: (i, 0)),
            pl.BlockSpec((_EMB_DIM, _NUM_EMB), lambda i: (0, 0)),
        ],
        out_specs=(
            pl.BlockSpec((_BM, 1), lambda i: (i, 0)),
            pl.BlockSpec((1, 1), lambda i: (0, 0)),
            pl.BlockSpec((1, _NUM_EMB), lambda i: (0, 0)),
            pl.BlockSpec((_NUM_EMB, _EMB_DIM), lambda i: (0, 0)),
        ),
        out_shape=(
            jax.ShapeDtypeStruct((_HALF, 1), jnp.int32),
            jax.ShapeDtypeStruct((1, 1), jnp.float32),
            jax.ShapeDtypeStruct((1, _NUM_EMB), jnp.float32),
            jax.ShapeDtypeStruct((_NUM_EMB, _EMB_DIM), jnp.float32),
        ),
        scratch_shapes=[
            pltpu.SMEM((1, 1), jnp.float32),
        ],
    )(xh, dictionary)


def _tc_half_b(xh, dictionary, a0, h0):
    grid = _HALF // _BM
    return pl.pallas_call(
        _tc_body_b,
        grid=(grid,),
        in_specs=[
            pl.BlockSpec((_BM, _EMB_DIM), lambda i: (i, 0)),
            pl.BlockSpec((_EMB_DIM, _NUM_EMB), lambda i: (0, 0)),
            pl.BlockSpec((1, 1), lambda i: (0, 0)),
            pl.BlockSpec((1, _NUM_EMB), lambda i: (0, 0)),
        ],
        out_specs=(
            pl.BlockSpec((_BM, 1), lambda i: (i, 0)),
            pl.BlockSpec((1, 1), lambda i: (0, 0)),
            pl.BlockSpec((1, 1), lambda i: (0, 0)),
        ),
        out_shape=(
            jax.ShapeDtypeStruct((_HALF, 1), jnp.int32),
            jax.ShapeDtypeStruct((1, 1), jnp.float32),
            jax.ShapeDtypeStruct((1, 1), jnp.float32),
        ),
        scratch_shapes=[
            pltpu.VMEM((1, _NUM_EMB), jnp.float32),
            pltpu.SMEM((1, 1), jnp.float32),
        ],
    )(xh, dictionary, a0, h0)


def _sc_gather(dict_t, idx2):
    mesh = plsc.VectorSubcoreMesh(core_axis_name="c", subcore_axis_name="s")
    n_rows = _NW * _BPW

    @functools.partial(
        pl.kernel,
        out_type=jax.ShapeDtypeStruct((n_rows, _EMB_DIM), jnp.float32),
        mesh=mesh,
        compiler_params=pltpu.CompilerParams(use_tc_tiling_on_sc=False),
        scratch_types=[
            pltpu.VMEM((_BPW,), jnp.int32),
            pltpu.VMEM((_BPW, _EMB_DIM), jnp.float32),
            pltpu.SemaphoreType.DMA,
        ],
    )
    def k(tab_hbm, idx_hbm, out_hbm, idx_v, rows_v, sem):
        c = lax.axis_index("c")
        s = lax.axis_index("s")
        wid = s * 2 + c
        pltpu.sync_copy(idx_hbm.at[wid], idx_v)
        copies = [
            pltpu.async_copy(tab_hbm.at[idx_v.at[pl.ds(j * 128, 128)]],
                             rows_v.at[pl.ds(j * 128, 128)], sem)
            for j in range(_BPW // 128)
        ]
        for cp in copies:
            cp.wait()
        pltpu.sync_copy(rows_v, out_hbm.at[pl.ds(wid * _BPW, _BPW)])

    return k(dict_t, idx2)


def kernel(x, dictionary):
    orig_shape = x.shape
    xf = x.reshape(-1, _EMB_DIM)
    idx0, a0, h0, dt = _tc_half_a(xf[:_HALF], dictionary)
    idx1, loss, perp = _tc_half_b(xf[_HALF:], dictionary, a0, h0)
    q0 = _sc_gather(dt, idx0.reshape(_NW, _BPW))
    q1 = _sc_gather(dt, idx1.reshape(_NW, _BPW))
    q = jnp.concatenate([q0, q1], axis=0)
    return q.reshape(orig_shape), loss[0, 0], perp[0, 0]


# speedup vs baseline: 1.2437x; 1.2437x over previous
import functools

import jax
import jax.numpy as jnp
import numpy as np
from jax import lax
from jax.experimental import pallas as pl
from jax.experimental.pallas import tpu as pltpu
from jax.experimental.pallas import tpu_sc as plsc

_NUM_EMB = 1024
_EMB_DIM = 64
_COM_COEF = 0.25
_BM = 1024
_NW = 32
_NROWS = 32768
_BPW = _NROWS // _NW


def _tc_body(x_ref, d_ref, idx_ref, loss_ref, perp_ref, dt_ref, hist, acc):
    i = pl.program_id(0)
    nsteps = pl.num_programs(0)
    xb = x_ref[...]
    dm = d_ref[...]
    sim = lax.dot_general(xb, dm, (((1,), (0,)), ((), ())),
                          preferred_element_type=jnp.float32)
    en2 = jnp.sum(dm * dm, axis=0, keepdims=True)
    dist = en2 - 2.0 * sim
    idx = jnp.argmin(dist, axis=1).astype(jnp.int32)
    idx_ref[...] = idx.reshape(_BM // 128, 128)
    m = jnp.min(dist, axis=1, keepdims=True)
    onehot = idx[:, None] == lax.broadcasted_iota(jnp.int32, (_BM, _NUM_EMB), 1)
    encf = onehot.astype(jnp.float32)
    ones_r = jnp.ones((1, _BM), jnp.float32)
    h = lax.dot_general(ones_r, encf, (((1,), (0,)), ((), ())),
                        preferred_element_type=jnp.float32)
    sq = xb * xb
    ones_c64 = jnp.ones((_EMB_DIM, 1), jnp.float32)
    xn2 = lax.dot_general(sq, ones_c64, (((1,), (0,)), ((), ())),
                          preferred_element_type=jnp.float32)
    row_min = m + xn2
    tot = lax.dot_general(ones_r, row_min, (((1,), (0,)), ((), ())),
                          preferred_element_type=jnp.float32)

    @pl.when(i == 0)
    def _():
        acc[0, 0] = 0.0
        hist[...] = jnp.zeros_like(hist)
        dt_ref[...] = lax.transpose(dm, (1, 0))

    acc[0, 0] += tot[0, 0]
    hist[...] += h

    @pl.when(i == nsteps - 1)
    def _():
        loss = (1.0 + _COM_COEF) * acc[0, 0] / (_NROWS * _EMB_DIM)
        loss_ref[...] = jnp.full((1, 1), loss, jnp.float32)
        p = hist[...] / _NROWS
        perp = jnp.exp(-jnp.sum(p * jnp.log(p + 1e-10)))
        perp_ref[...] = jnp.full((1, 1), perp, jnp.float32)


def _tc_argmin(xf, dictionary):
    n_rows = xf.shape[0]
    grid = n_rows // _BM
    rpb = _BM // 128
    return pl.pallas_call(
        _tc_body,
        grid=(grid,),
        in_specs=[
            pl.BlockSpec((_BM, _EMB_DIM), lambda i: (i, 0)),
            pl.BlockSpec((_EMB_DIM, _NUM_EMB), lambda i: (0, 0)),
        ],
        out_specs=(
            pl.BlockSpec((rpb, 128), lambda i: (i, 0)),
            pl.BlockSpec((1, 1), lambda i: (0, 0)),
            pl.BlockSpec((1, 1), lambda i: (0, 0)),
            pl.BlockSpec((_NUM_EMB, _EMB_DIM), lambda i: (0, 0)),
        ),
        out_shape=(
            jax.ShapeDtypeStruct((n_rows // 128, 128), jnp.int32),
            jax.ShapeDtypeStruct((1, 1), jnp.float32),
            jax.ShapeDtypeStruct((1, 1), jnp.float32),
            jax.ShapeDtypeStruct((_NUM_EMB, _EMB_DIM), jnp.float32),
        ),
        scratch_shapes=[
            pltpu.VMEM((1, _NUM_EMB), jnp.float32),
            pltpu.SMEM((1, 1), jnp.float32),
        ],
    )(xf, dictionary)


def _sc_gather(dict_t, idx2):
    mesh = plsc.VectorSubcoreMesh(core_axis_name="c", subcore_axis_name="s")
    rpw = _BPW // 128

    @functools.partial(
        pl.kernel,
        out_type=jax.ShapeDtypeStruct((_NROWS, _EMB_DIM), jnp.float32),
        mesh=mesh,
        compiler_params=pltpu.CompilerParams(use_tc_tiling_on_sc=False),
        scratch_types=[
            pltpu.VMEM((rpw, 128), jnp.int32),
            pltpu.VMEM((_BPW, _EMB_DIM), jnp.float32),
            pltpu.SemaphoreType.DMA,
        ],
    )
    def k(tab_hbm, idx_hbm, out_hbm, idx_v, rows_v, sem):
        c = lax.axis_index("c")
        s = lax.axis_index("s")
        wid = s * 2 + c
        pltpu.sync_copy(idx_hbm.at[pl.ds(wid * rpw, rpw)], idx_v)
        copies = [
            pltpu.async_copy(tab_hbm.at[idx_v.at[j]],
                             rows_v.at[pl.ds(j * 128, 128)], sem)
            for j in range(rpw)
        ]
        for cp in copies:
            cp.wait()
        pltpu.sync_copy(rows_v, out_hbm.at[pl.ds(wid * _BPW, _BPW)])

    return k(dict_t, idx2)


def kernel(x, dictionary):
    orig_shape = x.shape
    xf = x.reshape(-1, _EMB_DIM)
    idx2, loss, perp, dt = _tc_argmin(xf, dictionary)
    q = _sc_gather(dt, idx2)
    return q.reshape(orig_shape), loss[0, 0], perp[0, 0]


# speedup vs baseline: 1.3248x; 1.0652x over previous
import functools

import jax
import jax.numpy as jnp
import numpy as np
from jax import lax
from jax.experimental import pallas as pl
from jax.experimental.pallas import tpu as pltpu
from jax.experimental.pallas import tpu_sc as plsc

_NUM_EMB = 1024
_EMB_DIM = 64
_COM_COEF = 0.25
_BM = 2048
_NW = 32
_NROWS = 32768
_BPW = _NROWS // _NW


def _tc_body(x_ref, d_ref, idx_ref, loss_ref, perp_ref, dt_ref, hist, acc):
    i = pl.program_id(0)
    nsteps = pl.num_programs(0)
    xb = x_ref[...]
    dm = d_ref[...]
    sim = lax.dot_general(xb, dm, (((1,), (0,)), ((), ())),
                          preferred_element_type=jnp.float32)
    en2 = jnp.sum(dm * dm, axis=0, keepdims=True)
    dist = en2 - 2.0 * sim
    idx = jnp.argmin(dist, axis=1).astype(jnp.int32)
    idx_ref[...] = idx.reshape(_BM // 128, 128)
    m = jnp.min(dist, axis=1, keepdims=True)
    onehot = idx[:, None] == lax.broadcasted_iota(jnp.int32, (_BM, _NUM_EMB), 1)
    encf = onehot.astype(jnp.float32)
    ones_r = jnp.ones((1, _BM), jnp.float32)
    h = lax.dot_general(ones_r, encf, (((1,), (0,)), ((), ())),
                        preferred_element_type=jnp.float32)
    sq = xb * xb
    ones_c64 = jnp.ones((_EMB_DIM, 1), jnp.float32)
    xn2 = lax.dot_general(sq, ones_c64, (((1,), (0,)), ((), ())),
                          preferred_element_type=jnp.float32)
    row_min = m + xn2
    tot = lax.dot_general(ones_r, row_min, (((1,), (0,)), ((), ())),
                          preferred_element_type=jnp.float32)

    @pl.when(i == 0)
    def _():
        acc[0, 0] = 0.0
        hist[...] = jnp.zeros_like(hist)
        dt_ref[...] = lax.transpose(dm, (1, 0))

    acc[0, 0] += tot[0, 0]
    hist[...] += h

    @pl.when(i == nsteps - 1)
    def _():
        loss = (1.0 + _COM_COEF) * acc[0, 0] / (_NROWS * _EMB_DIM)
        loss_ref[...] = jnp.full((1, 1), loss, jnp.float32)
        p = hist[...] / _NROWS
        perp = jnp.exp(-jnp.sum(p * jnp.log(p + 1e-10)))
        perp_ref[...] = jnp.full((1, 1), perp, jnp.float32)


def _tc_argmin(xf, dictionary):
    n_rows = xf.shape[0]
    grid = n_rows // _BM
    rpb = _BM // 128
    return pl.pallas_call(
        _tc_body,
        grid=(grid,),
        in_specs=[
            pl.BlockSpec((_BM, _EMB_DIM), lambda i: (i, 0)),
            pl.BlockSpec((_EMB_DIM, _NUM_EMB), lambda i: (0, 0)),
        ],
        out_specs=(
            pl.BlockSpec((rpb, 128), lambda i: (i, 0)),
            pl.BlockSpec((1, 1), lambda i: (0, 0)),
            pl.BlockSpec((1, 1), lambda i: (0, 0)),
            pl.BlockSpec((_NUM_EMB, _EMB_DIM), lambda i: (0, 0)),
        ),
        out_shape=(
            jax.ShapeDtypeStruct((n_rows // 128, 128), jnp.int32),
            jax.ShapeDtypeStruct((1, 1), jnp.float32),
            jax.ShapeDtypeStruct((1, 1), jnp.float32),
            jax.ShapeDtypeStruct((_NUM_EMB, _EMB_DIM), jnp.float32),
        ),
        scratch_shapes=[
            pltpu.VMEM((1, _NUM_EMB), jnp.float32),
            pltpu.SMEM((1, 1), jnp.float32),
        ],
    )(xf, dictionary)


def _sc_gather(dict_t, idx2):
    mesh = plsc.VectorSubcoreMesh(core_axis_name="c", subcore_axis_name="s")
    rpw = _BPW // 128

    @functools.partial(
        pl.kernel,
        out_type=jax.ShapeDtypeStruct((_NROWS, _EMB_DIM), jnp.float32),
        mesh=mesh,
        compiler_params=pltpu.CompilerParams(use_tc_tiling_on_sc=False),
        scratch_types=[
            pltpu.VMEM((rpw, 128), jnp.int32),
            pltpu.VMEM((_BPW, _EMB_DIM), jnp.float32),
            pltpu.SemaphoreType.DMA,
        ],
    )
    def k(tab_hbm, idx_hbm, out_hbm, idx_v, rows_v, sem):
        c = lax.axis_index("c")
        s = lax.axis_index("s")
        wid = s * 2 + c
        pltpu.sync_copy(idx_hbm.at[pl.ds(wid * rpw, rpw)], idx_v)
        copies = [
            pltpu.async_copy(tab_hbm.at[idx_v.at[j]],
                             rows_v.at[pl.ds(j * 128, 128)], sem)
            for j in range(rpw)
        ]
        for cp in copies:
            cp.wait()
        pltpu.sync_copy(rows_v, out_hbm.at[pl.ds(wid * _BPW, _BPW)])

    return k(dict_t, idx2)


def kernel(x, dictionary):
    orig_shape = x.shape
    xf = x.reshape(-1, _EMB_DIM)
    idx2, loss, perp, dt = _tc_argmin(xf, dictionary)
    q = _sc_gather(dt, idx2)
    return q.reshape(orig_shape), loss[0, 0], perp[0, 0]


# speedup vs baseline: 1.3868x; 1.0469x over previous
import functools

import jax
import jax.numpy as jnp
import numpy as np
from jax import lax
from jax.experimental import pallas as pl
from jax.experimental.pallas import tpu as pltpu
from jax.experimental.pallas import tpu_sc as plsc

_NUM_EMB = 1024
_EMB_DIM = 64
_COM_COEF = 0.25
_BM = 4096
_NW = 32
_NROWS = 32768
_BPW = _NROWS // _NW


def _tc_body(x_ref, d_ref, idx_ref, loss_ref, perp_ref, dt_ref, hist, acc):
    i = pl.program_id(0)
    nsteps = pl.num_programs(0)
    xb = x_ref[...]
    dm = d_ref[...]
    sim = lax.dot_general(xb, dm, (((1,), (0,)), ((), ())),
                          preferred_element_type=jnp.float32)
    en2 = jnp.sum(dm * dm, axis=0, keepdims=True)
    dist = en2 - 2.0 * sim
    idx = jnp.argmin(dist, axis=1).astype(jnp.int32)
    idx_ref[...] = idx.reshape(_BM // 128, 128)
    m = jnp.min(dist, axis=1, keepdims=True)
    onehot = idx[:, None] == lax.broadcasted_iota(jnp.int32, (_BM, _NUM_EMB), 1)
    encf = onehot.astype(jnp.float32)
    ones_r = jnp.ones((1, _BM), jnp.float32)
    h = lax.dot_general(ones_r, encf, (((1,), (0,)), ((), ())),
                        preferred_element_type=jnp.float32)
    sq = xb * xb
    ones_c64 = jnp.ones((_EMB_DIM, 1), jnp.float32)
    xn2 = lax.dot_general(sq, ones_c64, (((1,), (0,)), ((), ())),
                          preferred_element_type=jnp.float32)
    row_min = m + xn2
    tot = lax.dot_general(ones_r, row_min, (((1,), (0,)), ((), ())),
                          preferred_element_type=jnp.float32)

    @pl.when(i == 0)
    def _():
        acc[0, 0] = 0.0
        hist[...] = jnp.zeros_like(hist)
        dt_ref[...] = lax.transpose(dm, (1, 0))

    acc[0, 0] += tot[0, 0]
    hist[...] += h

    @pl.when(i == nsteps - 1)
    def _():
        loss = (1.0 + _COM_COEF) * acc[0, 0] / (_NROWS * _EMB_DIM)
        loss_ref[...] = jnp.full((1, 1), loss, jnp.float32)
        p = hist[...] / _NROWS
        perp = jnp.exp(-jnp.sum(p * jnp.log(p + 1e-10)))
        perp_ref[...] = jnp.full((1, 1), perp, jnp.float32)


def _tc_argmin(xf, dictionary):
    n_rows = xf.shape[0]
    grid = n_rows // _BM
    rpb = _BM // 128
    return pl.pallas_call(
        _tc_body,
        grid=(grid,),
        in_specs=[
            pl.BlockSpec((_BM, _EMB_DIM), lambda i: (i, 0)),
            pl.BlockSpec((_EMB_DIM, _NUM_EMB), lambda i: (0, 0)),
        ],
        out_specs=(
            pl.BlockSpec((rpb, 128), lambda i: (i, 0)),
            pl.BlockSpec((1, 1), lambda i: (0, 0)),
            pl.BlockSpec((1, 1), lambda i: (0, 0)),
            pl.BlockSpec((_NUM_EMB, _EMB_DIM), lambda i: (0, 0)),
        ),
        out_shape=(
            jax.ShapeDtypeStruct((n_rows // 128, 128), jnp.int32),
            jax.ShapeDtypeStruct((1, 1), jnp.float32),
            jax.ShapeDtypeStruct((1, 1), jnp.float32),
            jax.ShapeDtypeStruct((_NUM_EMB, _EMB_DIM), jnp.float32),
        ),
        scratch_shapes=[
            pltpu.VMEM((1, _NUM_EMB), jnp.float32),
            pltpu.SMEM((1, 1), jnp.float32),
        ],
    )(xf, dictionary)


def _sc_gather(dict_t, idx2):
    mesh = plsc.VectorSubcoreMesh(core_axis_name="c", subcore_axis_name="s")
    rpw = _BPW // 128

    @functools.partial(
        pl.kernel,
        out_type=jax.ShapeDtypeStruct((_NROWS, _EMB_DIM), jnp.float32),
        mesh=mesh,
        compiler_params=pltpu.CompilerParams(use_tc_tiling_on_sc=False),
        scratch_types=[
            pltpu.VMEM((rpw, 128), jnp.int32),
            pltpu.VMEM((_BPW, _EMB_DIM), jnp.float32),
            pltpu.SemaphoreType.DMA,
        ],
    )
    def k(tab_hbm, idx_hbm, out_hbm, idx_v, rows_v, sem):
        c = lax.axis_index("c")
        s = lax.axis_index("s")
        wid = s * 2 + c
        pltpu.sync_copy(idx_hbm.at[pl.ds(wid * rpw, rpw)], idx_v)
        copies = [
            pltpu.async_copy(tab_hbm.at[idx_v.at[j]],
                             rows_v.at[pl.ds(j * 128, 128)], sem)
            for j in range(rpw)
        ]
        for cp in copies:
            cp.wait()
        pltpu.sync_copy(rows_v, out_hbm.at[pl.ds(wid * _BPW, _BPW)])

    return k(dict_t, idx2)


def kernel(x, dictionary):
    orig_shape = x.shape
    xf = x.reshape(-1, _EMB_DIM)
    idx2, loss, perp, dt = _tc_argmin(xf, dictionary)
    q = _sc_gather(dt, idx2)
    return q.reshape(orig_shape), loss[0, 0], perp[0, 0]


# speedup vs baseline: 1.4115x; 1.0178x over previous
import functools

import jax
import jax.numpy as jnp
import numpy as np
from jax import lax
from jax.experimental import pallas as pl
from jax.experimental.pallas import tpu as pltpu
from jax.experimental.pallas import tpu_sc as plsc

_NUM_EMB = 1024
_EMB_DIM = 64
_COM_COEF = 0.25
_BM = 8192
_NW = 32
_NROWS = 32768
_BPW = _NROWS // _NW


def _tc_body(x_ref, d_ref, idx_ref, loss_ref, perp_ref, dt_ref, hist, acc):
    i = pl.program_id(0)
    nsteps = pl.num_programs(0)
    xb = x_ref[...]
    dm = d_ref[...]
    sim = lax.dot_general(xb, dm, (((1,), (0,)), ((), ())),
                          preferred_element_type=jnp.float32)
    en2 = jnp.sum(dm * dm, axis=0, keepdims=True)
    dist = en2 - 2.0 * sim
    idx = jnp.argmin(dist, axis=1).astype(jnp.int32)
    idx_ref[...] = idx.reshape(_BM // 128, 128)
    m = jnp.min(dist, axis=1, keepdims=True)
    onehot = idx[:, None] == lax.broadcasted_iota(jnp.int32, (_BM, _NUM_EMB), 1)
    encf = onehot.astype(jnp.float32)
    ones_r = jnp.ones((1, _BM), jnp.float32)
    h = lax.dot_general(ones_r, encf, (((1,), (0,)), ((), ())),
                        preferred_element_type=jnp.float32)
    sq = xb * xb
    ones_c64 = jnp.ones((_EMB_DIM, 1), jnp.float32)
    xn2 = lax.dot_general(sq, ones_c64, (((1,), (0,)), ((), ())),
                          preferred_element_type=jnp.float32)
    row_min = m + xn2
    tot = lax.dot_general(ones_r, row_min, (((1,), (0,)), ((), ())),
                          preferred_element_type=jnp.float32)

    @pl.when(i == 0)
    def _():
        acc[0, 0] = 0.0
        hist[...] = jnp.zeros_like(hist)
        dt_ref[...] = lax.transpose(dm, (1, 0))

    acc[0, 0] += tot[0, 0]
    hist[...] += h

    @pl.when(i == nsteps - 1)
    def _():
        loss = (1.0 + _COM_COEF) * acc[0, 0] / (_NROWS * _EMB_DIM)
        loss_ref[...] = jnp.full((1, 1), loss, jnp.float32)
        p = hist[...] / _NROWS
        perp = jnp.exp(-jnp.sum(p * jnp.log(p + 1e-10)))
        perp_ref[...] = jnp.full((1, 1), perp, jnp.float32)


def _tc_argmin(xf, dictionary):
    n_rows = xf.shape[0]
    grid = n_rows // _BM
    rpb = _BM // 128
    return pl.pallas_call(
        _tc_body,
        grid=(grid,),
        in_specs=[
            pl.BlockSpec((_BM, _EMB_DIM), lambda i: (i, 0)),
            pl.BlockSpec((_EMB_DIM, _NUM_EMB), lambda i: (0, 0)),
        ],
        out_specs=(
            pl.BlockSpec((rpb, 128), lambda i: (i, 0)),
            pl.BlockSpec((1, 1), lambda i: (0, 0)),
            pl.BlockSpec((1, 1), lambda i: (0, 0)),
            pl.BlockSpec((_NUM_EMB, _EMB_DIM), lambda i: (0, 0)),
        ),
        out_shape=(
            jax.ShapeDtypeStruct((n_rows // 128, 128), jnp.int32),
            jax.ShapeDtypeStruct((1, 1), jnp.float32),
            jax.ShapeDtypeStruct((1, 1), jnp.float32),
            jax.ShapeDtypeStruct((_NUM_EMB, _EMB_DIM), jnp.float32),
        ),
        scratch_shapes=[
            pltpu.VMEM((1, _NUM_EMB), jnp.float32),
            pltpu.SMEM((1, 1), jnp.float32),
        ],
    )(xf, dictionary)


def _sc_gather(dict_t, idx2):
    mesh = plsc.VectorSubcoreMesh(core_axis_name="c", subcore_axis_name="s")
    rpw = _BPW // 128

    @functools.partial(
        pl.kernel,
        out_type=jax.ShapeDtypeStruct((_NROWS, _EMB_DIM), jnp.float32),
        mesh=mesh,
        compiler_params=pltpu.CompilerParams(use_tc_tiling_on_sc=False),
        scratch_types=[
            pltpu.VMEM((rpw, 128), jnp.int32),
            pltpu.VMEM((_BPW, _EMB_DIM), jnp.float32),
            pltpu.SemaphoreType.DMA,
        ],
    )
    def k(tab_hbm, idx_hbm, out_hbm, idx_v, rows_v, sem):
        c = lax.axis_index("c")
        s = lax.axis_index("s")
        wid = s * 2 + c
        pltpu.sync_copy(idx_hbm.at[pl.ds(wid * rpw, rpw)], idx_v)
        copies = [
            pltpu.async_copy(tab_hbm.at[idx_v.at[j]],
                             rows_v.at[pl.ds(j * 128, 128)], sem)
            for j in range(rpw)
        ]
        for cp in copies:
            cp.wait()
        pltpu.sync_copy(rows_v, out_hbm.at[pl.ds(wid * _BPW, _BPW)])

    return k(dict_t, idx2)


def kernel(x, dictionary):
    orig_shape = x.shape
    xf = x.reshape(-1, _EMB_DIM)
    idx2, loss, perp, dt = _tc_argmin(xf, dictionary)
    q = _sc_gather(dt, idx2)
    return q.reshape(orig_shape), loss[0, 0], perp[0, 0]
